# baseline (speedup 1.0000x reference)
import functools

import jax
import jax.numpy as jnp
from jax import lax
from jax.experimental import pallas as pl
from jax.experimental.pallas import tpu as pltpu
from jax.experimental.pallas import tpu_sc as plsc

N = 10000
D = 128
E = 320000
NC = 2
NS = 16
NW = NC * NS
EPW = E // NW
CH = 125
CPW = EPW // CH
HALF = CPW // 2
ECH = E // CH
NPAD = 10240
DEG_R = NPAD // D
STR = 624
NREM = N - NS * STR

_MESH = dict(core_axis_name="c", subcore_axis_name="s")


def _sc_deg_body(col_hbm, ew_hbm, deg_out,
                 col_v, ew_v, deg_v, acc_v, tmp_v, shared):
    c = lax.axis_index("c")
    s = lax.axis_index("s")
    base = (c * NS + s) * EPW

    pltpu.sync_copy(col_hbm.at[pl.ds(base, EPW)], col_v)
    pltpu.sync_copy(ew_hbm.at[pl.ds(base, EPW)], ew_v)

    def zero(i, _):
        deg_v[pl.ds(i * 16, 16)] = jnp.zeros((16,), jnp.float32)
        return 0
    lax.fori_loop(0, NPAD // 16, zero, 0)

    def body(i, _):
        cv = col_v[pl.ds(i * 16, 16)]
        wv = ew_v[pl.ds(i * 16, 16)]
        plsc.addupdate_scatter(deg_v, [cv], wv)
        return 0
    lax.fori_loop(0, EPW // 16, body, 0)

    pltpu.sync_copy(deg_v, shared.at[pl.ds(s * NPAD, NPAD)])
    plsc.subcore_barrier()

    sl = NPAD // NS
    off = s * sl
    pltpu.sync_copy(shared.at[pl.ds(off, sl)], acc_v)
    for w in range(1, NS):
        pltpu.sync_copy(shared.at[pl.ds(w * NPAD + off, sl)], tmp_v)

        def add(j, _):
            acc_v[pl.ds(j * 16, 16)] = \
                acc_v[pl.ds(j * 16, 16)] + tmp_v[pl.ds(j * 16, 16)]
            return 0
        lax.fori_loop(0, sl // 16, add, 0)

    pltpu.sync_copy(acc_v, deg_out.at[pl.ds(c * NPAD + off, sl)])


_sc_deg = functools.partial(
    pl.kernel,
    out_type=jax.ShapeDtypeStruct((NC * NPAD,), jnp.float32),
    mesh=plsc.VectorSubcoreMesh(num_cores=NC, num_subcores=NS, **_MESH),
    compiler_params=pltpu.CompilerParams(needs_layout_passes=False),
    scratch_types=[
        pltpu.VMEM((EPW,), jnp.int32),
        pltpu.VMEM((EPW,), jnp.float32),
        pltpu.VMEM((NPAD,), jnp.float32),
        pltpu.VMEM((NPAD // NS,), jnp.float32),
        pltpu.VMEM((NPAD // NS,), jnp.float32),
        pltpu.VMEM_SHARED((NS * NPAD,), jnp.float32),
    ],
)(_sc_deg_body)


def _sc_agg_body(xs_hbm, row_hbm, col_hbm, ew_hbm, zeros_hbm, out_hbm,
                 row_v, col_v, ew_v, buf0, buf1, sem0, sem1, shared):
    c = lax.axis_index("c")
    s = lax.axis_index("s")
    br = (c * NS + s) * CPW

    pltpu.sync_copy(zeros_hbm.at[pl.ds(s * STR, STR)],
                    shared.at[pl.ds(s * STR, STR)])

    @pl.when(s == NS - 1)
    def _():
        pltpu.sync_copy(zeros_hbm.at[pl.ds(NS * STR, NREM)],
                        shared.at[pl.ds(NS * STR, NREM)])

    plsc.subcore_barrier()

    def phase(start):
        pltpu.sync_copy(row_hbm.at[pl.ds(br + start, HALF)], row_v)
        pltpu.sync_copy(col_hbm.at[pl.ds(br + start, HALF)], col_v)
        pltpu.sync_copy(ew_hbm.at[pl.ds(br + start, HALF)], ew_v)

        pltpu.async_copy(xs_hbm.at[row_v.at[0]], buf0, sem0)
        pltpu.async_copy(xs_hbm.at[row_v.at[1]], buf1, sem1)

        def step(j, buf, sem):
            pltpu.make_async_copy(xs_hbm.at[row_v.at[j]], buf, sem).wait()
            splat_j = jnp.zeros((16,), jnp.int32) + j

            @plsc.parallel_loop(0, CH, unroll=25)
            def _(e):
                nv = plsc.load_gather(
                    ew_v, [splat_j, jnp.zeros((16,), jnp.int32) + e])
                for k in range(8):
                    buf[e, pl.ds(k * 16, 16)] = \
                        buf[e, pl.ds(k * 16, 16)] * nv
            pltpu.sync_copy(buf, shared.at[col_v.at[j]], add=True)

            @pl.when(j + 2 < HALF)
            def _():
                pltpu.async_copy(xs_hbm.at[row_v.at[j + 2]], buf, sem)

        def loop(i, _):
            step(i * 2, buf0, sem0)
            step(i * 2 + 1, buf1, sem1)
            return 0
        lax.fori_loop(0, HALF // 2, loop, 0)

    phase(0)
    phase(HALF)

    plsc.subcore_barrier()
    pltpu.sync_copy(shared.at[pl.ds(s * STR, STR)],
                    out_hbm.at[c, pl.ds(s * STR, STR)])

    @pl.when(s == NS - 1)
    def _():
        pltpu.sync_copy(shared.at[pl.ds(NS * STR, NREM)],
                        out_hbm.at[c, pl.ds(NS * STR, NREM)])


_sc_agg = functools.partial(
    pl.kernel,
    out_type=jax.ShapeDtypeStruct((NC, N, D), jnp.float32),
    mesh=plsc.VectorSubcoreMesh(num_cores=NC, num_subcores=NS, **_MESH),
    compiler_params=pltpu.CompilerParams(needs_layout_passes=False),
    scratch_types=[
        pltpu.VMEM((HALF, CH), jnp.int32),
        pltpu.VMEM((HALF, CH), jnp.int32),
        pltpu.VMEM((HALF, CH), jnp.float32),
        pltpu.VMEM((CH, D), jnp.float32),
        pltpu.VMEM((CH, D), jnp.float32),
        pltpu.SemaphoreType.DMA,
        pltpu.SemaphoreType.DMA,
        pltpu.VMEM_SHARED((N, D), jnp.float32),
    ],
)(_sc_agg_body)


BM = 1000


def _tc1_body(dega, degb, x, w1, xw_ref, xs_ref):
    dis = lax.rsqrt(dega[...] + degb[...] + 1.0)
    xw = jnp.dot(x[...], w1[...], preferred_element_type=jnp.float32)
    xw_ref[...] = xw
    xs_ref[...] = dis * xw


def _tc2_body(dega, degb, pa, pb, xw1, b1, w2, xw2_ref, xs2_ref):
    dis = lax.rsqrt(dega[...] + degb[...] + 1.0)
    h = dis * (pa[...] + pb[...]) + dis * dis * xw1[...] + b1[...]
    h = jnp.maximum(h, 0.0)
    xw2 = jnp.dot(h, w2[...], preferred_element_type=jnp.float32)
    xw2_ref[...] = xw2
    xs2_ref[...] = dis * xw2


def _tc3_body(dega, degb, pa, pb, xw2, b2, w3, b3, o_ref):
    dis = lax.rsqrt(dega[...] + degb[...] + 1.0)
    h = dis * (pa[...] + pb[...]) + dis * dis * xw2[...] + b2[...]
    h = jnp.maximum(h, 0.0)
    o_ref[...] = jnp.tanh(
        jnp.dot(h, w3[...], preferred_element_type=jnp.float32) + b3[...])


def _colspec():
    return pl.BlockSpec((BM, 1), lambda i: (i, 0))


def _rowspec():
    return pl.BlockSpec((BM, D), lambda i: (i, 0))


def _fullspec(r):
    return pl.BlockSpec((r, D), lambda i: (0, 0))


def _tc1(dega, degb, x, w1):
    return pl.pallas_call(
        _tc1_body,
        grid=(N // BM,),
        in_specs=[_colspec(), _colspec(), _rowspec(), _fullspec(D)],
        out_specs=[_rowspec(), _rowspec()],
        out_shape=[jax.ShapeDtypeStruct((N, D), jnp.float32)] * 2,
    )(dega, degb, x, w1)


def _tc2(dega, degb, pa, pb, xw1, b1, w2):
    return pl.pallas_call(
        _tc2_body,
        grid=(N // BM,),
        in_specs=[_colspec(), _colspec(), _rowspec(), _rowspec(), _rowspec(),
                  _fullspec(1), _fullspec(D)],
        out_specs=[_rowspec(), _rowspec()],
        out_shape=[jax.ShapeDtypeStruct((N, D), jnp.float32)] * 2,
    )(dega, degb, pa, pb, xw1, b1, w2)


def _tc3(dega, degb, pa, pb, xw2, b2, w3, b3):
    return pl.pallas_call(
        _tc3_body,
        grid=(N // BM,),
        in_specs=[_colspec(), _colspec(), _rowspec(), _rowspec(), _rowspec(),
                  _fullspec(1), _fullspec(D), _fullspec(1)],
        out_specs=_rowspec(),
        out_shape=jax.ShapeDtypeStruct((N, D), jnp.float32),
    )(dega, degb, pa, pb, xw2, b2, w3, b3)


def kernel(x, edge_index, edge_weight, W1, b1, W2, b2, W3, b3):
    row = edge_index[0].astype(jnp.int32)
    col = edge_index[1].astype(jnp.int32)
    ew = edge_weight.astype(jnp.float32)
    row2 = row.reshape(ECH, CH)
    col2 = col.reshape(ECH, CH)
    ew2 = ew.reshape(ECH, CH)
    zeros2d = jnp.zeros((N, D), jnp.float32)

    deg_parts = _sc_deg(col, ew).reshape(NC, NPAD)
    dega = deg_parts[0, :N].reshape(N, 1)
    degb = deg_parts[1, :N].reshape(N, 1)

    xw1, xs1 = _tc1(dega, degb, x, W1)
    parts1 = _sc_agg(xs1, row2, col2, ew2, zeros2d)
    xw2, xs2 = _tc2(dega, degb, parts1[0], parts1[1], xw1,
                    b1.reshape(1, D), W2)
    parts2 = _sc_agg(xs2, row2, col2, ew2, zeros2d)
    out = _tc3(dega, degb, parts2[0], parts2[1], xw2,
               b2.reshape(1, D), W3, b3.reshape(1, D))
    return out


# speedup vs baseline: 1.0202x; 1.0202x over previous
import functools

import jax
import jax.numpy as jnp
from jax import lax
from jax.experimental import pallas as pl
from jax.experimental.pallas import tpu as pltpu
from jax.experimental.pallas import tpu_sc as plsc

N = 10000
D = 128
E = 320000
NC = 2
NS = 16
NW = NC * NS
EPW = E // NW
CH = 125
CPW = EPW // CH
HALF = CPW // 2
ECH = E // CH
NPAD = 10240
DEG_R = NPAD // D
STR = 624
NREM = N - NS * STR

_MESH = dict(core_axis_name="c", subcore_axis_name="s")


def _sc_deg_body(col_hbm, ew_hbm, deg_out,
                 col_v, ew_v, deg_v, acc_v, tmp_v, shared):
    c = lax.axis_index("c")
    s = lax.axis_index("s")
    base = (c * NS + s) * EPW

    pltpu.sync_copy(col_hbm.at[pl.ds(base, EPW)], col_v)
    pltpu.sync_copy(ew_hbm.at[pl.ds(base, EPW)], ew_v)

    def zero(i, _):
        deg_v[pl.ds(i * 16, 16)] = jnp.zeros((16,), jnp.float32)
        return 0
    lax.fori_loop(0, NPAD // 16, zero, 0)

    @plsc.parallel_loop(0, EPW // 16, unroll=5)
    def _(i):
        cv = col_v[pl.ds(i * 16, 16)]
        wv = ew_v[pl.ds(i * 16, 16)]
        plsc.addupdate_scatter(deg_v, [cv], wv)

    pltpu.sync_copy(deg_v, shared.at[pl.ds(s * NPAD, NPAD)])
    plsc.subcore_barrier()

    sl = NPAD // NS
    off = s * sl
    pltpu.sync_copy(shared.at[pl.ds(off, sl)], acc_v)
    for w in range(1, NS):
        pltpu.sync_copy(shared.at[pl.ds(w * NPAD + off, sl)], tmp_v)

        def add(j, _):
            acc_v[pl.ds(j * 16, 16)] = \
                acc_v[pl.ds(j * 16, 16)] + tmp_v[pl.ds(j * 16, 16)]
            return 0
        lax.fori_loop(0, sl // 16, add, 0)

    pltpu.sync_copy(acc_v, deg_out.at[pl.ds(c * NPAD + off, sl)])


_sc_deg = functools.partial(
    pl.kernel,
    out_type=jax.ShapeDtypeStruct((NC * NPAD,), jnp.float32),
    mesh=plsc.VectorSubcoreMesh(num_cores=NC, num_subcores=NS, **_MESH),
    compiler_params=pltpu.CompilerParams(needs_layout_passes=False),
    scratch_types=[
        pltpu.VMEM((EPW,), jnp.int32),
        pltpu.VMEM((EPW,), jnp.float32),
        pltpu.VMEM((NPAD,), jnp.float32),
        pltpu.VMEM((NPAD // NS,), jnp.float32),
        pltpu.VMEM((NPAD // NS,), jnp.float32),
        pltpu.VMEM_SHARED((NS * NPAD,), jnp.float32),
    ],
)(_sc_deg_body)


def _sc_agg_body(xs_hbm, row_hbm, col_hbm, ew_hbm, zeros_hbm, out_hbm,
                 row_v, col_v, ew_v, buf0, buf1, sem0, sem1, sem_s, shared):
    c = lax.axis_index("c")
    s = lax.axis_index("s")
    br = (c * NS + s) * CPW

    pltpu.sync_copy(zeros_hbm.at[pl.ds(s * STR, STR)],
                    shared.at[pl.ds(s * STR, STR)])

    @pl.when(s == NS - 1)
    def _():
        pltpu.sync_copy(zeros_hbm.at[pl.ds(NS * STR, NREM)],
                        shared.at[pl.ds(NS * STR, NREM)])

    plsc.subcore_barrier()

    def phase(start):
        pltpu.sync_copy(row_hbm.at[pl.ds(br + start, HALF)], row_v)
        pltpu.sync_copy(col_hbm.at[pl.ds(br + start, HALF)], col_v)
        pltpu.sync_copy(ew_hbm.at[pl.ds(br + start, HALF)], ew_v)

        pltpu.async_copy(xs_hbm.at[row_v.at[0]], buf0, sem0)
        pltpu.async_copy(xs_hbm.at[row_v.at[1]], buf1, sem1)

        def step(j, buf, sem, sem_s):
            pltpu.make_async_copy(xs_hbm.at[row_v.at[j]], buf, sem).wait()
            splat_j = jnp.zeros((16,), jnp.int32) + j

            for g in range(5):
                @plsc.parallel_loop(g * 25, (g + 1) * 25, unroll=5)
                def _(e):
                    nv = plsc.load_gather(
                        ew_v, [splat_j, jnp.zeros((16,), jnp.int32) + e])
                    for k in range(8):
                        buf[e, pl.ds(k * 16, 16)] = \
                            buf[e, pl.ds(k * 16, 16)] * nv
                pltpu.async_copy(
                    buf.at[pl.ds(g * 25, 25)],
                    shared.at[col_v.at[j, pl.ds(g * 25, 25)]],
                    sem_s, add=True)
            for g in range(5):
                pltpu.make_async_copy(
                    buf.at[pl.ds(g * 25, 25)],
                    shared.at[col_v.at[j, pl.ds(g * 25, 25)]],
                    sem_s).wait()

            @pl.when(j + 2 < HALF)
            def _():
                pltpu.async_copy(xs_hbm.at[row_v.at[j + 2]], buf, sem)

        def loop(i, _):
            step(i * 2, buf0, sem0, sem_s)
            step(i * 2 + 1, buf1, sem1, sem_s)
            return 0
        lax.fori_loop(0, HALF // 2, loop, 0)

    phase(0)
    phase(HALF)

    plsc.subcore_barrier()
    pltpu.sync_copy(shared.at[pl.ds(s * STR, STR)],
                    out_hbm.at[c, pl.ds(s * STR, STR)])

    @pl.when(s == NS - 1)
    def _():
        pltpu.sync_copy(shared.at[pl.ds(NS * STR, NREM)],
                        out_hbm.at[c, pl.ds(NS * STR, NREM)])


_sc_agg = functools.partial(
    pl.kernel,
    out_type=jax.ShapeDtypeStruct((NC, N, D), jnp.float32),
    mesh=plsc.VectorSubcoreMesh(num_cores=NC, num_subcores=NS, **_MESH),
    compiler_params=pltpu.CompilerParams(needs_layout_passes=False),
    scratch_types=[
        pltpu.VMEM((HALF, CH), jnp.int32),
        pltpu.VMEM((HALF, CH), jnp.int32),
        pltpu.VMEM((HALF, CH), jnp.float32),
        pltpu.VMEM((CH, D), jnp.float32),
        pltpu.VMEM((CH, D), jnp.float32),
        pltpu.SemaphoreType.DMA,
        pltpu.SemaphoreType.DMA,
        pltpu.SemaphoreType.DMA,
        pltpu.VMEM_SHARED((N, D), jnp.float32),
    ],
)(_sc_agg_body)


BM = 1000


def _tc1_body(dega, degb, x, w1, xw_ref, xs_ref):
    dis = lax.rsqrt(dega[...] + degb[...] + 1.0)
    xw = jnp.dot(x[...], w1[...], preferred_element_type=jnp.float32)
    xw_ref[...] = xw
    xs_ref[...] = dis * xw


def _tc2_body(dega, degb, pa, pb, xw1, b1, w2, xw2_ref, xs2_ref):
    dis = lax.rsqrt(dega[...] + degb[...] + 1.0)
    h = dis * (pa[...] + pb[...]) + dis * dis * xw1[...] + b1[...]
    h = jnp.maximum(h, 0.0)
    xw2 = jnp.dot(h, w2[...], preferred_element_type=jnp.float32)
    xw2_ref[...] = xw2
    xs2_ref[...] = dis * xw2


def _tc3_body(dega, degb, pa, pb, xw2, b2, w3, b3, o_ref):
    dis = lax.rsqrt(dega[...] + degb[...] + 1.0)
    h = dis * (pa[...] + pb[...]) + dis * dis * xw2[...] + b2[...]
    h = jnp.maximum(h, 0.0)
    o_ref[...] = jnp.tanh(
        jnp.dot(h, w3[...], preferred_element_type=jnp.float32) + b3[...])


def _colspec():
    return pl.BlockSpec((BM, 1), lambda i: (i, 0))


def _rowspec():
    return pl.BlockSpec((BM, D), lambda i: (i, 0))


def _fullspec(r):
    return pl.BlockSpec((r, D), lambda i: (0, 0))


def _tc1(dega, degb, x, w1):
    return pl.pallas_call(
        _tc1_body,
        grid=(N // BM,),
        in_specs=[_colspec(), _colspec(), _rowspec(), _fullspec(D)],
        out_specs=[_rowspec(), _rowspec()],
        out_shape=[jax.ShapeDtypeStruct((N, D), jnp.float32)] * 2,
    )(dega, degb, x, w1)


def _tc2(dega, degb, pa, pb, xw1, b1, w2):
    return pl.pallas_call(
        _tc2_body,
        grid=(N // BM,),
        in_specs=[_colspec(), _colspec(), _rowspec(), _rowspec(), _rowspec(),
                  _fullspec(1), _fullspec(D)],
        out_specs=[_rowspec(), _rowspec()],
        out_shape=[jax.ShapeDtypeStruct((N, D), jnp.float32)] * 2,
    )(dega, degb, pa, pb, xw1, b1, w2)


def _tc3(dega, degb, pa, pb, xw2, b2, w3, b3):
    return pl.pallas_call(
        _tc3_body,
        grid=(N // BM,),
        in_specs=[_colspec(), _colspec(), _rowspec(), _rowspec(), _rowspec(),
                  _fullspec(1), _fullspec(D), _fullspec(1)],
        out_specs=_rowspec(),
        out_shape=jax.ShapeDtypeStruct((N, D), jnp.float32),
    )(dega, degb, pa, pb, xw2, b2, w3, b3)


def kernel(x, edge_index, edge_weight, W1, b1, W2, b2, W3, b3):
    row = edge_index[0].astype(jnp.int32)
    col = edge_index[1].astype(jnp.int32)
    ew = edge_weight.astype(jnp.float32)
    row2 = row.reshape(ECH, CH)
    col2 = col.reshape(ECH, CH)
    ew2 = ew.reshape(ECH, CH)
    zeros2d = jnp.zeros((N, D), jnp.float32)

    deg_parts = _sc_deg(col, ew).reshape(NC, NPAD)
    dega = deg_parts[0, :N].reshape(N, 1)
    degb = deg_parts[1, :N].reshape(N, 1)

    xw1, xs1 = _tc1(dega, degb, x, W1)
    parts1 = _sc_agg(xs1, row2, col2, ew2, zeros2d)
    xw2, xs2 = _tc2(dega, degb, parts1[0], parts1[1], xw1,
                    b1.reshape(1, D), W2)
    parts2 = _sc_agg(xs2, row2, col2, ew2, zeros2d)
    out = _tc3(dega, degb, parts2[0], parts2[1], xw2,
               b2.reshape(1, D), W3, b3.reshape(1, D))
    return out
